# baseline (speedup 1.0000x reference)
import jax
import jax.numpy as jnp
from jax.experimental import pallas as pl
from jax.experimental.pallas import tpu as pltpu

NUM_PREPEND = 16


def _prepend_body(x_ref, emb_ref, out_ref, sems):
    B = x_ref.shape[0]
    S = x_ref.shape[1]
    copies = []
    for b in range(B):
        copies.append(
            pltpu.make_async_copy(
                x_ref.at[b],
                out_ref.at[b, pl.ds(NUM_PREPEND, S)],
                sems.at[2 * b],
            )
        )
        copies.append(
            pltpu.make_async_copy(
                emb_ref,
                out_ref.at[b, pl.ds(0, NUM_PREPEND)],
                sems.at[2 * b + 1],
            )
        )
    for c in copies:
        c.start()
    for c in copies:
        c.wait()


def kernel(x, embed_table):
    B, S, D = x.shape
    out_shape = jax.ShapeDtypeStruct((B, S + NUM_PREPEND, D), x.dtype)
    return pl.pallas_call(
        _prepend_body,
        out_shape=out_shape,
        in_specs=[
            pl.BlockSpec(memory_space=pltpu.MemorySpace.HBM),
            pl.BlockSpec(memory_space=pltpu.MemorySpace.HBM),
        ],
        out_specs=pl.BlockSpec(memory_space=pltpu.MemorySpace.HBM),
        scratch_shapes=[pltpu.SemaphoreType.DMA((2 * B,))],
    )(x, embed_table)


# speedup vs baseline: 47.4091x; 47.4091x over previous
import jax
import jax.numpy as jnp
from jax.experimental import pallas as pl
from jax.experimental.pallas import tpu as pltpu

NUM_PREPEND = 16
CHUNK_ROWS = 256
NBUF = 8
DEPTH = 4


def _prepend_body(x_hbm, emb_hbm, out_hbm, buf, emb_v,
                  ld_sems, st_sems, esem, tsems):
    B, S, D = x_hbm.shape
    per_batch = S // CHUNK_ROWS
    nch = B * per_batch

    emb_load = pltpu.make_async_copy(emb_hbm, emb_v, esem)
    emb_load.start()

    loads = []
    stores = []
    for i in range(nch):
        b, c = divmod(i, per_batch)
        j = i % NBUF
        loads.append(pltpu.make_async_copy(
            x_hbm.at[b, pl.ds(c * CHUNK_ROWS, CHUNK_ROWS)],
            buf.at[j], ld_sems.at[j]))
        stores.append(pltpu.make_async_copy(
            buf.at[j],
            out_hbm.at[b, pl.ds(NUM_PREPEND + c * CHUNK_ROWS, CHUNK_ROWS)],
            st_sems.at[j]))

    for i in range(nch + DEPTH):
        if i < nch:
            if i >= NBUF:
                stores[i - NBUF].wait()
            loads[i].start()
        if i == 0:
            emb_load.wait()
            for b in range(B):
                pltpu.make_async_copy(
                    emb_v, out_hbm.at[b, pl.ds(0, NUM_PREPEND)], tsems.at[b]
                ).start()
        k = i - DEPTH
        if 0 <= k < nch:
            loads[k].wait()
            stores[k].start()

    for i in range(nch - NBUF, nch):
        stores[i].wait()
    for b in range(B):
        pltpu.make_async_copy(
            emb_v, out_hbm.at[b, pl.ds(0, NUM_PREPEND)], tsems.at[b]
        ).wait()


def kernel(x, embed_table):
    B, S, D = x.shape
    out_shape = jax.ShapeDtypeStruct((B, S + NUM_PREPEND, D), x.dtype)
    return pl.pallas_call(
        _prepend_body,
        out_shape=out_shape,
        in_specs=[
            pl.BlockSpec(memory_space=pltpu.MemorySpace.HBM),
            pl.BlockSpec(memory_space=pltpu.MemorySpace.HBM),
        ],
        out_specs=pl.BlockSpec(memory_space=pltpu.MemorySpace.HBM),
        scratch_shapes=[
            pltpu.VMEM((NBUF, CHUNK_ROWS, D), x.dtype),
            pltpu.VMEM((NUM_PREPEND, D), embed_table.dtype),
            pltpu.SemaphoreType.DMA((NBUF,)),
            pltpu.SemaphoreType.DMA((NBUF,)),
            pltpu.SemaphoreType.DMA,
            pltpu.SemaphoreType.DMA((B,)),
        ],
    )(x, embed_table)


# speedup vs baseline: 48.1889x; 1.0164x over previous
import jax
import jax.numpy as jnp
from jax.experimental import pallas as pl
from jax.experimental.pallas import tpu as pltpu

NUM_PREPEND = 16
CHUNK_ROWS = 512
NBUF = 8
DEPTH = 6


def _prepend_body(x_hbm, emb_hbm, out_hbm, buf, emb_v,
                  ld_sems, st_sems, esem, tsems):
    B, S, D = x_hbm.shape
    per_batch = S // CHUNK_ROWS
    nch = B * per_batch

    emb_load = pltpu.make_async_copy(emb_hbm, emb_v, esem)
    emb_load.start()

    loads = []
    stores = []
    for i in range(nch):
        b, c = divmod(i, per_batch)
        j = i % NBUF
        loads.append(pltpu.make_async_copy(
            x_hbm.at[b, pl.ds(c * CHUNK_ROWS, CHUNK_ROWS)],
            buf.at[j], ld_sems.at[j]))
        stores.append(pltpu.make_async_copy(
            buf.at[j],
            out_hbm.at[b, pl.ds(NUM_PREPEND + c * CHUNK_ROWS, CHUNK_ROWS)],
            st_sems.at[j]))

    for i in range(nch + DEPTH):
        if i < nch:
            if i >= NBUF:
                stores[i - NBUF].wait()
            loads[i].start()
        if i == 0:
            emb_load.wait()
            for b in range(B):
                pltpu.make_async_copy(
                    emb_v, out_hbm.at[b, pl.ds(0, NUM_PREPEND)], tsems.at[b]
                ).start()
        k = i - DEPTH
        if 0 <= k < nch:
            loads[k].wait()
            stores[k].start()

    for i in range(nch - NBUF, nch):
        stores[i].wait()
    for b in range(B):
        pltpu.make_async_copy(
            emb_v, out_hbm.at[b, pl.ds(0, NUM_PREPEND)], tsems.at[b]
        ).wait()


def kernel(x, embed_table):
    B, S, D = x.shape
    out_shape = jax.ShapeDtypeStruct((B, S + NUM_PREPEND, D), x.dtype)
    return pl.pallas_call(
        _prepend_body,
        out_shape=out_shape,
        in_specs=[
            pl.BlockSpec(memory_space=pltpu.MemorySpace.HBM),
            pl.BlockSpec(memory_space=pltpu.MemorySpace.HBM),
        ],
        out_specs=pl.BlockSpec(memory_space=pltpu.MemorySpace.HBM),
        scratch_shapes=[
            pltpu.VMEM((NBUF, CHUNK_ROWS, D), x.dtype),
            pltpu.VMEM((NUM_PREPEND, D), embed_table.dtype),
            pltpu.SemaphoreType.DMA((NBUF,)),
            pltpu.SemaphoreType.DMA((NBUF,)),
            pltpu.SemaphoreType.DMA,
            pltpu.SemaphoreType.DMA((B,)),
        ],
    )(x, embed_table)
